# Initial kernel scaffold; baseline (speedup 1.0000x reference)
import jax
import jax.numpy as jnp
from jax.experimental import pallas as pl


def kernel(x, edge_index, batch, W1_rel, b1, W1_root, W2_rel, b2, W2_root, W3_rel, b3, W3_root, Wlin, blin):
    raise NotImplementedError("write your pallas kernel here")


# speedup vs baseline: 5.0321x; 5.0321x over previous
import functools

import jax
import jax.numpy as jnp
from jax import lax
from jax.experimental import pallas as pl
from jax.experimental.pallas import tpu as pltpu
from jax.experimental.pallas import tpu_sc as plsc

N = 10000
D = 128
G = 64
NC = 2
NS = 16
NW = NC * NS
E_CHUNK = 80
N_PAD = 10240
ROWS_PER_TILE = N_PAD // NS
ROW_BLK = 1000
N_BLKS = N // ROW_BLK


def _mm2_body(x_ref, wa_ref, wb_ref, b_ref, ya_ref, yb_ref):
    x = x_ref[...]
    ya_ref[...] = jnp.dot(x, wa_ref[...], preferred_element_type=jnp.float32)
    yb_ref[...] = jnp.dot(x, wb_ref[...], preferred_element_type=jnp.float32) + b_ref[...]


def _dual_matmul(x, wa, wb, b_row):
    return pl.pallas_call(
        _mm2_body,
        grid=(N_BLKS,),
        in_specs=[
            pl.BlockSpec((ROW_BLK, D), lambda i: (i, 0)),
            pl.BlockSpec((D, D), lambda i: (0, 0)),
            pl.BlockSpec((D, D), lambda i: (0, 0)),
            pl.BlockSpec((1, D), lambda i: (0, 0)),
        ],
        out_specs=[
            pl.BlockSpec((ROW_BLK, D), lambda i: (i, 0)),
            pl.BlockSpec((ROW_BLK, D), lambda i: (i, 0)),
        ],
        out_shape=[jax.ShapeDtypeStruct((N, D), jnp.float32)] * 2,
    )(x, wa, wb, b_row)


def _combine_body(p_ref, r_ref, wa_ref, wb_ref, b_ref, ya_ref, yb_ref):
    h = jnp.maximum(p_ref[0] + p_ref[1] + r_ref[...], 0.0)
    ya_ref[...] = jnp.dot(h, wa_ref[...], preferred_element_type=jnp.float32)
    yb_ref[...] = jnp.dot(h, wb_ref[...], preferred_element_type=jnp.float32) + b_ref[...]


def _combine_matmul(p, r, wa, wb, b_row):
    return pl.pallas_call(
        _combine_body,
        grid=(N_BLKS,),
        in_specs=[
            pl.BlockSpec((NC, ROW_BLK, D), lambda i: (0, i, 0)),
            pl.BlockSpec((ROW_BLK, D), lambda i: (i, 0)),
            pl.BlockSpec((D, D), lambda i: (0, 0)),
            pl.BlockSpec((D, D), lambda i: (0, 0)),
            pl.BlockSpec((1, D), lambda i: (0, 0)),
        ],
        out_specs=[
            pl.BlockSpec((ROW_BLK, D), lambda i: (i, 0)),
            pl.BlockSpec((ROW_BLK, D), lambda i: (i, 0)),
        ],
        out_shape=[jax.ShapeDtypeStruct((N, D), jnp.float32)] * 2,
    )(p, r, wa, wb, b_row)


def _final_body(p_ref, r_ref, batch_ref, wlin_ref, blin_ref, out_ref, sums, counts):
    i = pl.program_id(0)

    @pl.when(i == 0)
    def _():
        sums[...] = jnp.zeros_like(sums)
        counts[...] = jnp.zeros_like(counts)

    h = p_ref[0] + p_ref[1] + r_ref[...]
    b = batch_ref[0]
    gid = lax.broadcasted_iota(jnp.int32, (G, ROW_BLK), 0)
    onehot = (gid == b).astype(jnp.float32)
    sums[...] += jnp.dot(onehot, h, preferred_element_type=jnp.float32)
    counts[...] = counts[...] + jnp.sum(onehot, axis=1, keepdims=True)

    @pl.when(i == pl.num_programs(0) - 1)
    def _():
        pooled = sums[...] / jnp.maximum(counts[...], 1.0)
        out_ref[...] = jnp.dot(pooled, wlin_ref[...],
                               preferred_element_type=jnp.float32) + blin_ref[...]


def _final_pool(p, r, batch3, wlin_pad, blin_row):
    return pl.pallas_call(
        _final_body,
        grid=(N_BLKS,),
        in_specs=[
            pl.BlockSpec((NC, ROW_BLK, D), lambda i: (0, i, 0)),
            pl.BlockSpec((ROW_BLK, D), lambda i: (i, 0)),
            pl.BlockSpec((1, 1, ROW_BLK), lambda i: (i, 0, 0)),
            pl.BlockSpec((D, D), lambda i: (0, 0)),
            pl.BlockSpec((1, D), lambda i: (0, 0)),
        ],
        out_specs=pl.BlockSpec((G, D), lambda i: (0, 0)),
        out_shape=jax.ShapeDtypeStruct((G, D), jnp.float32),
        scratch_shapes=[
            pltpu.VMEM((G, D), jnp.float32),
            pltpu.VMEM((G, D), jnp.float32),
        ],
        compiler_params=pltpu.CompilerParams(
            dimension_semantics=("arbitrary",)),
    )(p, r, batch3, wlin_pad, blin_row)


def _sc_scatter(y, src, dst, zeros):
    E = src.shape[0]
    epw = E // NW
    n_chunks = epw // E_CHUNK

    mesh = plsc.VectorSubcoreMesh(
        core_axis_name="c", subcore_axis_name="s",
        num_cores=NC, num_subcores=NS)

    @functools.partial(
        pl.kernel,
        out_type=jax.ShapeDtypeStruct((NC * N_PAD, D), jnp.float32),
        mesh=mesh,
        scratch_types=[
            pltpu.VMEM((E_CHUNK,), jnp.int32),
            pltpu.VMEM((E_CHUNK,), jnp.int32),
            pltpu.VMEM((E_CHUNK, D), jnp.float32),
            pltpu.VMEM_SHARED((N_PAD, D), jnp.float32),
            pltpu.SemaphoreType.DMA,
        ],
    )
    def k(y_hbm, src_hbm, dst_hbm, zeros_hbm, out_hbm, src_v, dst_v, rows_v, acc, sem):
        c = lax.axis_index("c")
        s = lax.axis_index("s")
        wid = c * NS + s
        row0 = s * ROWS_PER_TILE

        pltpu.sync_copy(zeros_hbm, acc.at[pl.ds(row0, ROWS_PER_TILE)])
        plsc.subcore_barrier()

        base0 = wid * epw

        def body(i, carry):
            base = base0 + i * E_CHUNK
            pltpu.sync_copy(src_hbm.at[pl.ds(base, E_CHUNK)], src_v)
            pltpu.sync_copy(dst_hbm.at[pl.ds(base, E_CHUNK)], dst_v)
            pltpu.async_copy(y_hbm.at[src_v], rows_v, sem).wait()
            pltpu.sync_copy(rows_v, acc.at[dst_v], add=True)
            return carry

        lax.fori_loop(0, n_chunks, body, 0)
        plsc.subcore_barrier()

        pltpu.sync_copy(acc.at[pl.ds(row0, ROWS_PER_TILE)],
                        out_hbm.at[pl.ds(c * N_PAD + row0, ROWS_PER_TILE)])

    return k(y, src, dst, zeros)


def kernel(x, edge_index, batch, W1_rel, b1, W1_root, W2_rel, b2, W2_root,
           W3_rel, b3, W3_root, Wlin, blin):
    src = edge_index[0]
    dst = edge_index[1]
    zeros = jnp.zeros((ROWS_PER_TILE, D), jnp.float32)
    b1r = b1.reshape(1, D)
    b2r = b2.reshape(1, D)
    b3r = b3.reshape(1, D)
    batch3 = batch.reshape(N_BLKS, 1, ROW_BLK)
    wlin_pad = jnp.zeros((D, D), jnp.float32).at[:, : Wlin.shape[1]].set(Wlin)
    blin_row = jnp.zeros((1, D), jnp.float32).at[0, : blin.shape[0]].set(blin)

    y1, r1 = _dual_matmul(x, W1_rel, W1_root, b1r)
    p1 = _sc_scatter(y1, src, dst, zeros).reshape(NC, N_PAD, D)
    y2, r2 = _combine_matmul(p1, r1, W2_rel, W2_root, b2r)
    p2 = _sc_scatter(y2, src, dst, zeros).reshape(NC, N_PAD, D)
    y3, r3 = _combine_matmul(p2, r2, W3_rel, W3_root, b3r)
    p3 = _sc_scatter(y3, src, dst, zeros).reshape(NC, N_PAD, D)
    out = _final_pool(p3, r3, batch3, wlin_pad, blin_row)
    return out[:, : Wlin.shape[1]]


# speedup vs baseline: 7.0199x; 1.3950x over previous
import functools

import jax
import jax.numpy as jnp
from jax import lax
from jax.experimental import pallas as pl
from jax.experimental.pallas import tpu as pltpu
from jax.experimental.pallas import tpu_sc as plsc

N = 10000
D = 128
G = 64
NC = 2
NS = 16
NW = NC * NS
E_CHUNK = 40
N_PAD = 10240
ROWS_PER_TILE = N_PAD // NS
ROW_BLK = 1000
N_BLKS = N // ROW_BLK


def _mm2_body(x_ref, wa_ref, wb_ref, b_ref, ya_ref, yb_ref):
    x = x_ref[...]
    ya_ref[...] = jnp.dot(x, wa_ref[...], preferred_element_type=jnp.float32)
    yb_ref[...] = jnp.dot(x, wb_ref[...], preferred_element_type=jnp.float32) + b_ref[...]


def _dual_matmul(x, wa, wb, b_row):
    return pl.pallas_call(
        _mm2_body,
        grid=(N_BLKS,),
        in_specs=[
            pl.BlockSpec((ROW_BLK, D), lambda i: (i, 0)),
            pl.BlockSpec((D, D), lambda i: (0, 0)),
            pl.BlockSpec((D, D), lambda i: (0, 0)),
            pl.BlockSpec((1, D), lambda i: (0, 0)),
        ],
        out_specs=[
            pl.BlockSpec((ROW_BLK, D), lambda i: (i, 0)),
            pl.BlockSpec((ROW_BLK, D), lambda i: (i, 0)),
        ],
        out_shape=[jax.ShapeDtypeStruct((N, D), jnp.float32)] * 2,
    )(x, wa, wb, b_row)


def _combine_body(p_ref, r_ref, wa_ref, wb_ref, b_ref, ya_ref, yb_ref):
    h = jnp.maximum(p_ref[0] + p_ref[1] + r_ref[...], 0.0)
    ya_ref[...] = jnp.dot(h, wa_ref[...], preferred_element_type=jnp.float32)
    yb_ref[...] = jnp.dot(h, wb_ref[...], preferred_element_type=jnp.float32) + b_ref[...]


def _combine_matmul(p, r, wa, wb, b_row):
    return pl.pallas_call(
        _combine_body,
        grid=(N_BLKS,),
        in_specs=[
            pl.BlockSpec((NC, ROW_BLK, D), lambda i: (0, i, 0)),
            pl.BlockSpec((ROW_BLK, D), lambda i: (i, 0)),
            pl.BlockSpec((D, D), lambda i: (0, 0)),
            pl.BlockSpec((D, D), lambda i: (0, 0)),
            pl.BlockSpec((1, D), lambda i: (0, 0)),
        ],
        out_specs=[
            pl.BlockSpec((ROW_BLK, D), lambda i: (i, 0)),
            pl.BlockSpec((ROW_BLK, D), lambda i: (i, 0)),
        ],
        out_shape=[jax.ShapeDtypeStruct((N, D), jnp.float32)] * 2,
    )(p, r, wa, wb, b_row)


def _final_body(p_ref, r_ref, batch_ref, wlin_ref, blin_ref, out_ref, sums, counts):
    i = pl.program_id(0)

    @pl.when(i == 0)
    def _():
        sums[...] = jnp.zeros_like(sums)
        counts[...] = jnp.zeros_like(counts)

    h = p_ref[0] + p_ref[1] + r_ref[...]
    b = batch_ref[0]
    gid = lax.broadcasted_iota(jnp.int32, (G, ROW_BLK), 0)
    onehot = (gid == b).astype(jnp.float32)
    sums[...] += jnp.dot(onehot, h, preferred_element_type=jnp.float32)
    counts[...] = counts[...] + jnp.sum(onehot, axis=1, keepdims=True)

    @pl.when(i == pl.num_programs(0) - 1)
    def _():
        pooled = sums[...] / jnp.maximum(counts[...], 1.0)
        out_ref[...] = jnp.dot(pooled, wlin_ref[...],
                               preferred_element_type=jnp.float32) + blin_ref[...]


def _final_pool(p, r, batch3, wlin_pad, blin_row):
    return pl.pallas_call(
        _final_body,
        grid=(N_BLKS,),
        in_specs=[
            pl.BlockSpec((NC, ROW_BLK, D), lambda i: (0, i, 0)),
            pl.BlockSpec((ROW_BLK, D), lambda i: (i, 0)),
            pl.BlockSpec((1, 1, ROW_BLK), lambda i: (i, 0, 0)),
            pl.BlockSpec((D, D), lambda i: (0, 0)),
            pl.BlockSpec((1, D), lambda i: (0, 0)),
        ],
        out_specs=pl.BlockSpec((G, D), lambda i: (0, 0)),
        out_shape=jax.ShapeDtypeStruct((G, D), jnp.float32),
        scratch_shapes=[
            pltpu.VMEM((G, D), jnp.float32),
            pltpu.VMEM((G, D), jnp.float32),
        ],
        compiler_params=pltpu.CompilerParams(
            dimension_semantics=("arbitrary",)),
    )(p, r, batch3, wlin_pad, blin_row)


def _sc_scatter(y, src, dst, zeros):
    E = src.shape[0]
    epw = E // NW
    n_chunks = epw // E_CHUNK

    mesh = plsc.VectorSubcoreMesh(
        core_axis_name="c", subcore_axis_name="s",
        num_cores=NC, num_subcores=NS)

    @functools.partial(
        pl.kernel,
        out_type=jax.ShapeDtypeStruct((NC * N_PAD, D), jnp.float32),
        mesh=mesh,
        scratch_types=[
            pltpu.VMEM((E_CHUNK,), jnp.int32),
            pltpu.VMEM((E_CHUNK,), jnp.int32),
            pltpu.VMEM((E_CHUNK,), jnp.int32),
            pltpu.VMEM((E_CHUNK,), jnp.int32),
            pltpu.VMEM((E_CHUNK, D), jnp.float32),
            pltpu.VMEM((E_CHUNK, D), jnp.float32),
            pltpu.VMEM_SHARED((N_PAD, D), jnp.float32),
            pltpu.SemaphoreType.DMA,
            pltpu.SemaphoreType.DMA,
            pltpu.SemaphoreType.DMA,
            pltpu.SemaphoreType.DMA,
        ],
    )
    def k(y_hbm, src_hbm, dst_hbm, zeros_hbm, out_hbm,
          src0, dst0, src1, dst1, rows0, rows1, acc, si0, si1, sg0, sg1):
        c = lax.axis_index("c")
        s = lax.axis_index("s")
        wid = c * NS + s
        row0 = s * ROWS_PER_TILE

        pltpu.sync_copy(zeros_hbm, acc.at[pl.ds(row0, ROWS_PER_TILE)])
        plsc.subcore_barrier()

        base0 = wid * epw

        def idx_start(i, sv, dv, sem):
            b = base0 + i * E_CHUNK
            pltpu.async_copy(src_hbm.at[pl.ds(b, E_CHUNK)], sv, sem)
            pltpu.async_copy(dst_hbm.at[pl.ds(b, E_CHUNK)], dv, sem)

        def idx_wait(sv, dv, sem):
            pltpu.make_async_copy(src_hbm.at[pl.ds(0, E_CHUNK)], sv, sem).wait()
            pltpu.make_async_copy(dst_hbm.at[pl.ds(0, E_CHUNK)], dv, sem).wait()

        def g_start(sv, rows, sem):
            pltpu.async_copy(y_hbm.at[sv], rows, sem)

        def g_wait(sv, rows, sem):
            pltpu.make_async_copy(y_hbm.at[sv], rows, sem).wait()

        def scat(dv, rows):
            pltpu.sync_copy(rows, acc.at[dv], add=True)

        idx_start(0, src0, dst0, si0)
        idx_wait(src0, dst0, si0)
        g_start(src0, rows0, sg0)
        idx_start(1, src1, dst1, si1)

        def body(j, carry):
            i = 2 * j
            idx_wait(src1, dst1, si1)
            g_start(src1, rows1, sg1)
            g_wait(src0, rows0, sg0)
            scat(dst0, rows0)

            @pl.when(i + 2 < n_chunks)
            def _():
                idx_start(i + 2, src0, dst0, si0)

            @pl.when(i + 2 < n_chunks)
            def _():
                idx_wait(src0, dst0, si0)
                g_start(src0, rows0, sg0)

            g_wait(src1, rows1, sg1)
            scat(dst1, rows1)

            @pl.when(i + 3 < n_chunks)
            def _():
                idx_start(i + 3, src1, dst1, si1)

            return carry

        lax.fori_loop(0, n_chunks // 2, body, 0)
        plsc.subcore_barrier()

        pltpu.sync_copy(acc.at[pl.ds(row0, ROWS_PER_TILE)],
                        out_hbm.at[pl.ds(c * N_PAD + row0, ROWS_PER_TILE)])

    return k(y, src, dst, zeros)


def kernel(x, edge_index, batch, W1_rel, b1, W1_root, W2_rel, b2, W2_root,
           W3_rel, b3, W3_root, Wlin, blin):
    src = edge_index[0]
    dst = edge_index[1]
    zeros = jnp.zeros((ROWS_PER_TILE, D), jnp.float32)
    b1r = b1.reshape(1, D)
    b2r = b2.reshape(1, D)
    b3r = b3.reshape(1, D)
    batch3 = batch.reshape(N_BLKS, 1, ROW_BLK)
    wlin_pad = jnp.zeros((D, D), jnp.float32).at[:, : Wlin.shape[1]].set(Wlin)
    blin_row = jnp.zeros((1, D), jnp.float32).at[0, : blin.shape[0]].set(blin)

    y1, r1 = _dual_matmul(x, W1_rel, W1_root, b1r)
    p1 = _sc_scatter(y1, src, dst, zeros).reshape(NC, N_PAD, D)
    y2, r2 = _combine_matmul(p1, r1, W2_rel, W2_root, b2r)
    p2 = _sc_scatter(y2, src, dst, zeros).reshape(NC, N_PAD, D)
    y3, r3 = _combine_matmul(p2, r2, W3_rel, W3_root, b3r)
    p3 = _sc_scatter(y3, src, dst, zeros).reshape(NC, N_PAD, D)
    out = _final_pool(p3, r3, batch3, wlin_pad, blin_row)
    return out[:, : Wlin.shape[1]]


# speedup vs baseline: 9.7545x; 1.3895x over previous
import functools

import jax
import jax.numpy as jnp
from jax import lax
from jax.experimental import pallas as pl
from jax.experimental.pallas import tpu as pltpu
from jax.experimental.pallas import tpu_sc as plsc

N = 10000
D = 128
G = 64
NC = 2
NS = 16
NW = NC * NS
E_CHUNK = 80
N_PAD = 10240
ROWS_PER_TILE = N_PAD // NS
ROW_BLK = 1000
N_BLKS = N // ROW_BLK


def _mm2_body(x_ref, wa_ref, wb_ref, b_ref, ya_ref, yb_ref):
    x = x_ref[...]
    ya_ref[...] = jnp.dot(x, wa_ref[...], preferred_element_type=jnp.float32)
    yb_ref[...] = jnp.dot(x, wb_ref[...], preferred_element_type=jnp.float32) + b_ref[...]


def _dual_matmul(x, wa, wb, b_row):
    return pl.pallas_call(
        _mm2_body,
        grid=(N_BLKS,),
        in_specs=[
            pl.BlockSpec((ROW_BLK, D), lambda i: (i, 0)),
            pl.BlockSpec((D, D), lambda i: (0, 0)),
            pl.BlockSpec((D, D), lambda i: (0, 0)),
            pl.BlockSpec((1, D), lambda i: (0, 0)),
        ],
        out_specs=[
            pl.BlockSpec((ROW_BLK, D), lambda i: (i, 0)),
            pl.BlockSpec((ROW_BLK, D), lambda i: (i, 0)),
        ],
        out_shape=[jax.ShapeDtypeStruct((N, D), jnp.float32)] * 2,
    )(x, wa, wb, b_row)


def _combine_body(p_ref, r_ref, wa_ref, wb_ref, b_ref, ya_ref, yb_ref):
    h = jnp.maximum(p_ref[0] + p_ref[1] + r_ref[...], 0.0)
    ya_ref[...] = jnp.dot(h, wa_ref[...], preferred_element_type=jnp.float32)
    yb_ref[...] = jnp.dot(h, wb_ref[...], preferred_element_type=jnp.float32) + b_ref[...]


def _combine_matmul(p, r, wa, wb, b_row):
    return pl.pallas_call(
        _combine_body,
        grid=(N_BLKS,),
        in_specs=[
            pl.BlockSpec((NC, ROW_BLK, D), lambda i: (0, i, 0)),
            pl.BlockSpec((ROW_BLK, D), lambda i: (i, 0)),
            pl.BlockSpec((D, D), lambda i: (0, 0)),
            pl.BlockSpec((D, D), lambda i: (0, 0)),
            pl.BlockSpec((1, D), lambda i: (0, 0)),
        ],
        out_specs=[
            pl.BlockSpec((ROW_BLK, D), lambda i: (i, 0)),
            pl.BlockSpec((ROW_BLK, D), lambda i: (i, 0)),
        ],
        out_shape=[jax.ShapeDtypeStruct((N, D), jnp.float32)] * 2,
    )(p, r, wa, wb, b_row)


def _final_body(p_ref, r_ref, batch_ref, wlin_ref, blin_ref, out_ref, sums, counts):
    i = pl.program_id(0)

    @pl.when(i == 0)
    def _():
        sums[...] = jnp.zeros_like(sums)
        counts[...] = jnp.zeros_like(counts)

    h = p_ref[0] + p_ref[1] + r_ref[...]
    b = batch_ref[0]
    gid = lax.broadcasted_iota(jnp.int32, (G, ROW_BLK), 0)
    onehot = (gid == b).astype(jnp.float32)
    sums[...] += jnp.dot(onehot, h, preferred_element_type=jnp.float32)
    counts[...] = counts[...] + jnp.sum(onehot, axis=1, keepdims=True)

    @pl.when(i == pl.num_programs(0) - 1)
    def _():
        pooled = sums[...] / jnp.maximum(counts[...], 1.0)
        out_ref[...] = jnp.dot(pooled, wlin_ref[...],
                               preferred_element_type=jnp.float32) + blin_ref[...]


def _final_pool(p, r, batch3, wlin_pad, blin_row):
    return pl.pallas_call(
        _final_body,
        grid=(N_BLKS,),
        in_specs=[
            pl.BlockSpec((NC, ROW_BLK, D), lambda i: (0, i, 0)),
            pl.BlockSpec((ROW_BLK, D), lambda i: (i, 0)),
            pl.BlockSpec((1, 1, ROW_BLK), lambda i: (i, 0, 0)),
            pl.BlockSpec((D, D), lambda i: (0, 0)),
            pl.BlockSpec((1, D), lambda i: (0, 0)),
        ],
        out_specs=pl.BlockSpec((G, D), lambda i: (0, 0)),
        out_shape=jax.ShapeDtypeStruct((G, D), jnp.float32),
        scratch_shapes=[
            pltpu.VMEM((G, D), jnp.float32),
            pltpu.VMEM((G, D), jnp.float32),
        ],
        compiler_params=pltpu.CompilerParams(
            dimension_semantics=("arbitrary",)),
    )(p, r, batch3, wlin_pad, blin_row)


def _sc_scatter(y, src, dst, zeros):
    E = src.shape[0]
    epw = E // NW
    n_chunks = epw // E_CHUNK

    mesh = plsc.VectorSubcoreMesh(
        core_axis_name="c", subcore_axis_name="s",
        num_cores=NC, num_subcores=NS)

    @functools.partial(
        pl.kernel,
        out_type=jax.ShapeDtypeStruct((NC * N_PAD, D), jnp.float32),
        mesh=mesh,
        scratch_types=[
            pltpu.VMEM((E_CHUNK,), jnp.int32),
            pltpu.VMEM((E_CHUNK,), jnp.int32),
            pltpu.VMEM((E_CHUNK,), jnp.int32),
            pltpu.VMEM((E_CHUNK,), jnp.int32),
            pltpu.VMEM((E_CHUNK, D), jnp.float32),
            pltpu.VMEM((E_CHUNK, D), jnp.float32),
            pltpu.VMEM_SHARED((N_PAD, D), jnp.float32),
            pltpu.SemaphoreType.DMA,
            pltpu.SemaphoreType.DMA,
            pltpu.SemaphoreType.DMA,
            pltpu.SemaphoreType.DMA,
        ],
    )
    def k(y_hbm, src_hbm, dst_hbm, zeros_hbm, out_hbm,
          src0, dst0, src1, dst1, rows0, rows1, acc, si0, si1, sg0, sg1):
        c = lax.axis_index("c")
        s = lax.axis_index("s")
        wid = c * NS + s
        row0 = s * ROWS_PER_TILE

        pltpu.sync_copy(zeros_hbm, acc.at[pl.ds(row0, ROWS_PER_TILE)])
        plsc.subcore_barrier()

        base0 = wid * epw

        def idx_start(i, sv, dv, sem):
            b = base0 + i * E_CHUNK
            pltpu.async_copy(src_hbm.at[pl.ds(b, E_CHUNK)], sv, sem)
            pltpu.async_copy(dst_hbm.at[pl.ds(b, E_CHUNK)], dv, sem)

        def idx_wait(sv, dv, sem):
            pltpu.make_async_copy(src_hbm.at[pl.ds(0, E_CHUNK)], sv, sem).wait()
            pltpu.make_async_copy(dst_hbm.at[pl.ds(0, E_CHUNK)], dv, sem).wait()

        def g_start(sv, rows, sem):
            pltpu.async_copy(y_hbm.at[sv], rows, sem)

        def g_wait(sv, rows, sem):
            pltpu.make_async_copy(y_hbm.at[sv], rows, sem).wait()

        def scat(dv, rows):
            pltpu.sync_copy(rows, acc.at[dv], add=True)

        idx_start(0, src0, dst0, si0)
        idx_wait(src0, dst0, si0)
        g_start(src0, rows0, sg0)
        idx_start(1, src1, dst1, si1)

        def body(j, carry):
            i = 2 * j
            idx_wait(src1, dst1, si1)
            g_start(src1, rows1, sg1)
            g_wait(src0, rows0, sg0)
            scat(dst0, rows0)

            @pl.when(i + 2 < n_chunks)
            def _():
                idx_start(i + 2, src0, dst0, si0)

            @pl.when(i + 2 < n_chunks)
            def _():
                idx_wait(src0, dst0, si0)
                g_start(src0, rows0, sg0)

            g_wait(src1, rows1, sg1)
            scat(dst1, rows1)

            @pl.when(i + 3 < n_chunks)
            def _():
                idx_start(i + 3, src1, dst1, si1)

            return carry

        lax.fori_loop(0, n_chunks // 2, body, 0)

        if n_chunks % 2 == 1:
            g_wait(src0, rows0, sg0)
            scat(dst0, rows0)

        plsc.subcore_barrier()

        pltpu.sync_copy(acc.at[pl.ds(row0, ROWS_PER_TILE)],
                        out_hbm.at[pl.ds(c * N_PAD + row0, ROWS_PER_TILE)])

    return k(y, src, dst, zeros)


def kernel(x, edge_index, batch, W1_rel, b1, W1_root, W2_rel, b2, W2_root,
           W3_rel, b3, W3_root, Wlin, blin):
    src = edge_index[0]
    dst = edge_index[1]
    zeros = jnp.zeros((ROWS_PER_TILE, D), jnp.float32)
    b1r = b1.reshape(1, D)
    b2r = b2.reshape(1, D)
    b3r = b3.reshape(1, D)
    batch3 = batch.reshape(N_BLKS, 1, ROW_BLK)
    wlin_pad = jnp.zeros((D, D), jnp.float32).at[:, : Wlin.shape[1]].set(Wlin)
    blin_row = jnp.zeros((1, D), jnp.float32).at[0, : blin.shape[0]].set(blin)

    y1, r1 = _dual_matmul(x, W1_rel, W1_root, b1r)
    p1 = _sc_scatter(y1, src, dst, zeros).reshape(NC, N_PAD, D)
    y2, r2 = _combine_matmul(p1, r1, W2_rel, W2_root, b2r)
    p2 = _sc_scatter(y2, src, dst, zeros).reshape(NC, N_PAD, D)
    y3, r3 = _combine_matmul(p2, r2, W3_rel, W3_root, b3r)
    p3 = _sc_scatter(y3, src, dst, zeros).reshape(NC, N_PAD, D)
    out = _final_pool(p3, r3, batch3, wlin_pad, blin_row)
    return out[:, : Wlin.shape[1]]


# speedup vs baseline: 11.1657x; 1.1447x over previous
import functools

import jax
import jax.numpy as jnp
from jax import lax
from jax.experimental import pallas as pl
from jax.experimental.pallas import tpu as pltpu
from jax.experimental.pallas import tpu_sc as plsc

N = 10000
D = 128
G = 64
NC = 2
NS = 16
NW = NC * NS
E_CHUNK = 128
N_PAD = 10240
ROWS_PER_TILE = N_PAD // NS
ROW_BLK = 1000
N_BLKS = N // ROW_BLK


def _mm2_body(x_ref, wa_ref, wb_ref, b_ref, ya_ref, yb_ref):
    x = x_ref[...]
    ya_ref[...] = jnp.dot(x, wa_ref[...], preferred_element_type=jnp.float32)
    yb_ref[...] = jnp.dot(x, wb_ref[...], preferred_element_type=jnp.float32) + b_ref[...]


def _dual_matmul(x, wa, wb, b_row):
    return pl.pallas_call(
        _mm2_body,
        grid=(N_BLKS,),
        in_specs=[
            pl.BlockSpec((ROW_BLK, D), lambda i: (i, 0)),
            pl.BlockSpec((D, D), lambda i: (0, 0)),
            pl.BlockSpec((D, D), lambda i: (0, 0)),
            pl.BlockSpec((1, D), lambda i: (0, 0)),
        ],
        out_specs=[
            pl.BlockSpec((ROW_BLK, D), lambda i: (i, 0)),
            pl.BlockSpec((ROW_BLK, D), lambda i: (i, 0)),
        ],
        out_shape=[jax.ShapeDtypeStruct((N, D), jnp.float32)] * 2,
    )(x, wa, wb, b_row)


def _combine_body(p_ref, r_ref, wa_ref, wb_ref, b_ref, ya_ref, yb_ref):
    h = jnp.maximum(p_ref[0] + p_ref[1] + r_ref[...], 0.0)
    ya_ref[...] = jnp.dot(h, wa_ref[...], preferred_element_type=jnp.float32)
    yb_ref[...] = jnp.dot(h, wb_ref[...], preferred_element_type=jnp.float32) + b_ref[...]


def _combine_matmul(p, r, wa, wb, b_row):
    return pl.pallas_call(
        _combine_body,
        grid=(N_BLKS,),
        in_specs=[
            pl.BlockSpec((NC, ROW_BLK, D), lambda i: (0, i, 0)),
            pl.BlockSpec((ROW_BLK, D), lambda i: (i, 0)),
            pl.BlockSpec((D, D), lambda i: (0, 0)),
            pl.BlockSpec((D, D), lambda i: (0, 0)),
            pl.BlockSpec((1, D), lambda i: (0, 0)),
        ],
        out_specs=[
            pl.BlockSpec((ROW_BLK, D), lambda i: (i, 0)),
            pl.BlockSpec((ROW_BLK, D), lambda i: (i, 0)),
        ],
        out_shape=[jax.ShapeDtypeStruct((N, D), jnp.float32)] * 2,
    )(p, r, wa, wb, b_row)


def _final_body(p_ref, r_ref, batch_ref, wlin_ref, blin_ref, out_ref, sums, counts):
    i = pl.program_id(0)

    @pl.when(i == 0)
    def _():
        sums[...] = jnp.zeros_like(sums)
        counts[...] = jnp.zeros_like(counts)

    h = p_ref[0] + p_ref[1] + r_ref[...]
    b = batch_ref[0]
    gid = lax.broadcasted_iota(jnp.int32, (G, ROW_BLK), 0)
    onehot = (gid == b).astype(jnp.float32)
    sums[...] += jnp.dot(onehot, h, preferred_element_type=jnp.float32)
    counts[...] = counts[...] + jnp.sum(onehot, axis=1, keepdims=True)

    @pl.when(i == pl.num_programs(0) - 1)
    def _():
        pooled = sums[...] / jnp.maximum(counts[...], 1.0)
        out_ref[...] = jnp.dot(pooled, wlin_ref[...],
                               preferred_element_type=jnp.float32) + blin_ref[...]


def _final_pool(p, r, batch3, wlin_pad, blin_row):
    return pl.pallas_call(
        _final_body,
        grid=(N_BLKS,),
        in_specs=[
            pl.BlockSpec((NC, ROW_BLK, D), lambda i: (0, i, 0)),
            pl.BlockSpec((ROW_BLK, D), lambda i: (i, 0)),
            pl.BlockSpec((1, 1, ROW_BLK), lambda i: (i, 0, 0)),
            pl.BlockSpec((D, D), lambda i: (0, 0)),
            pl.BlockSpec((1, D), lambda i: (0, 0)),
        ],
        out_specs=pl.BlockSpec((G, D), lambda i: (0, 0)),
        out_shape=jax.ShapeDtypeStruct((G, D), jnp.float32),
        scratch_shapes=[
            pltpu.VMEM((G, D), jnp.float32),
            pltpu.VMEM((G, D), jnp.float32),
        ],
        compiler_params=pltpu.CompilerParams(
            dimension_semantics=("arbitrary",)),
    )(p, r, batch3, wlin_pad, blin_row)


def _sc_scatter(y, src, dst, zeros):
    E = src.shape[0]
    epw = E // NW
    n_chunks = epw // E_CHUNK
    tail = epw % E_CHUNK

    mesh = plsc.VectorSubcoreMesh(
        core_axis_name="c", subcore_axis_name="s",
        num_cores=NC, num_subcores=NS)

    @functools.partial(
        pl.kernel,
        out_type=jax.ShapeDtypeStruct((NC * N_PAD, D), jnp.float32),
        mesh=mesh,
        scratch_types=[
            pltpu.VMEM((E_CHUNK,), jnp.int32),
            pltpu.VMEM((E_CHUNK,), jnp.int32),
            pltpu.VMEM((E_CHUNK,), jnp.int32),
            pltpu.VMEM((E_CHUNK,), jnp.int32),
            pltpu.VMEM((E_CHUNK, D), jnp.float32),
            pltpu.VMEM((E_CHUNK, D), jnp.float32),
            pltpu.VMEM((max(tail, 8),), jnp.int32),
            pltpu.VMEM((max(tail, 8),), jnp.int32),
            pltpu.VMEM_SHARED((N_PAD, D), jnp.float32),
            pltpu.SemaphoreType.DMA,
            pltpu.SemaphoreType.DMA,
            pltpu.SemaphoreType.DMA,
            pltpu.SemaphoreType.DMA,
        ],
    )
    def k(y_hbm, src_hbm, dst_hbm, zeros_hbm, out_hbm,
          src0, dst0, src1, dst1, rows0, rows1, src_t, dst_t, acc,
          si0, si1, sg0, sg1):
        c = lax.axis_index("c")
        s = lax.axis_index("s")
        wid = c * NS + s
        row0 = s * ROWS_PER_TILE

        pltpu.sync_copy(zeros_hbm, acc.at[pl.ds(row0, ROWS_PER_TILE)])
        plsc.subcore_barrier()

        base0 = wid * epw

        def idx_start(i, sv, dv, sem):
            b = base0 + i * E_CHUNK
            pltpu.async_copy(src_hbm.at[pl.ds(b, E_CHUNK)], sv, sem)
            pltpu.async_copy(dst_hbm.at[pl.ds(b, E_CHUNK)], dv, sem)

        def idx_wait(sv, dv, sem):
            pltpu.make_async_copy(src_hbm.at[pl.ds(0, E_CHUNK)], sv, sem).wait()
            pltpu.make_async_copy(dst_hbm.at[pl.ds(0, E_CHUNK)], dv, sem).wait()

        def g_start(sv, rows, sem):
            pltpu.async_copy(y_hbm.at[sv], rows, sem)

        def g_wait(sv, rows, sem):
            pltpu.make_async_copy(y_hbm.at[sv], rows, sem).wait()

        def scat(dv, rows):
            pltpu.sync_copy(rows, acc.at[dv], add=True)

        idx_start(0, src0, dst0, si0)
        idx_wait(src0, dst0, si0)
        g_start(src0, rows0, sg0)
        idx_start(1, src1, dst1, si1)

        def body(j, carry):
            i = 2 * j
            idx_wait(src1, dst1, si1)
            g_start(src1, rows1, sg1)
            g_wait(src0, rows0, sg0)
            scat(dst0, rows0)

            @pl.when(i + 2 < n_chunks)
            def _():
                idx_start(i + 2, src0, dst0, si0)

            @pl.when(i + 2 < n_chunks)
            def _():
                idx_wait(src0, dst0, si0)
                g_start(src0, rows0, sg0)

            g_wait(src1, rows1, sg1)
            scat(dst1, rows1)

            @pl.when(i + 3 < n_chunks)
            def _():
                idx_start(i + 3, src1, dst1, si1)

            return carry

        lax.fori_loop(0, n_chunks // 2, body, 0)

        if n_chunks % 2 == 1:
            g_wait(src0, rows0, sg0)
            scat(dst0, rows0)

        if tail:
            tb = base0 + n_chunks * E_CHUNK
            pltpu.sync_copy(src_hbm.at[pl.ds(tb, tail)], src_t)
            pltpu.sync_copy(dst_hbm.at[pl.ds(tb, tail)], dst_t)
            rows_t = rows0.at[pl.ds(0, tail)]
            pltpu.async_copy(y_hbm.at[src_t], rows_t, sg0).wait()
            pltpu.sync_copy(rows_t, acc.at[dst_t], add=True)

        plsc.subcore_barrier()

        pltpu.sync_copy(acc.at[pl.ds(row0, ROWS_PER_TILE)],
                        out_hbm.at[pl.ds(c * N_PAD + row0, ROWS_PER_TILE)])

    return k(y, src, dst, zeros)


def kernel(x, edge_index, batch, W1_rel, b1, W1_root, W2_rel, b2, W2_root,
           W3_rel, b3, W3_root, Wlin, blin):
    src = edge_index[0]
    dst = edge_index[1]
    zeros = jnp.zeros((ROWS_PER_TILE, D), jnp.float32)
    b1r = b1.reshape(1, D)
    b2r = b2.reshape(1, D)
    b3r = b3.reshape(1, D)
    batch3 = batch.reshape(N_BLKS, 1, ROW_BLK)
    wlin_pad = jnp.zeros((D, D), jnp.float32).at[:, : Wlin.shape[1]].set(Wlin)
    blin_row = jnp.zeros((1, D), jnp.float32).at[0, : blin.shape[0]].set(blin)

    y1, r1 = _dual_matmul(x, W1_rel, W1_root, b1r)
    p1 = _sc_scatter(y1, src, dst, zeros).reshape(NC, N_PAD, D)
    y2, r2 = _combine_matmul(p1, r1, W2_rel, W2_root, b2r)
    p2 = _sc_scatter(y2, src, dst, zeros).reshape(NC, N_PAD, D)
    y3, r3 = _combine_matmul(p2, r2, W3_rel, W3_root, b3r)
    p3 = _sc_scatter(y3, src, dst, zeros).reshape(NC, N_PAD, D)
    out = _final_pool(p3, r3, batch3, wlin_pad, blin_row)
    return out[:, : Wlin.shape[1]]
